# baseline (speedup 1.0000x reference)
import functools

import jax
import jax.numpy as jnp
from jax import lax
from jax.experimental import pallas as pl
from jax.experimental.pallas import tpu as pltpu
from jax.experimental.pallas import tpu_sc as plsc

VOCAB = 100000
DIM = 128
P = 200
N = 16384

NC = 2
NS = 16
NW = NC * NS
L = 16

N_PER_W = N // NW
N_CHUNKS = N_PER_W // 128
P_PER_W = 8
P_WORKERS = P // P_PER_W

D_VREGS = DIM // L


def _sigmoid(v):
  return 1.0 / (1.0 + jnp.exp(-v))


def _dot16(rows_ref, row_base, wv, out_ref, out_base):
  lane = lax.iota(jnp.int32, L)
  fifteen = jnp.full((L,), L - 1, jnp.int32)
  out_v = jnp.zeros((L,), jnp.float32)
  for r in range(L):
    row = row_base + r
    p = [rows_ref[row, pl.ds(j * L, L)] * wv[j] for j in range(D_VREGS)]
    while len(p) > 1:
      p = [p[i] + p[i + 1] for i in range(0, len(p), 2)]
    scanned = plsc.cumsum(p[0])
    tot = lax.gather(
        scanned, fifteen[:, None],
        lax.GatherDimensionNumbers(
            offset_dims=(), collapsed_slice_dims=(0,), start_index_map=(0,)),
        slice_sizes=(1,),
        mode=lax.GatherScatterMode.PROMISE_IN_BOUNDS)
    out_v = jnp.where(lane == r, tot, out_v)
  out_ref[pl.ds(out_base, L)] = _sigmoid(out_v)


def _w2v_body(word_hbm, pos_hbm, neg_hbm, wemb_hbm, cemb_hbm,
              out_p_hbm, out_n_hbm,
              word_v, wrow_v, idx_n_v, rows_n_v, idx_p_v, rows_p_v,
              out_n_v, out_p_v, sem, psem, wsem):
  wid = lax.axis_index("s") * NC + lax.axis_index("c")

  pltpu.sync_copy(word_hbm, word_v)
  pltpu.async_copy(wemb_hbm.at[word_v], wrow_v, wsem).wait()

  pltpu.sync_copy(neg_hbm.at[pl.ds(wid * N_CHUNKS, N_CHUNKS)], idx_n_v)

  copies = []
  for j in range(N_CHUNKS):
    copies.append(pltpu.async_copy(
        cemb_hbm.at[idx_n_v.at[j]], rows_n_v.at[pl.ds(j * 128, 128)], sem))

  @pl.when(wid < P_WORKERS)
  def _():
    pltpu.sync_copy(pos_hbm.at[pl.ds(wid * P_PER_W, P_PER_W)], idx_p_v)
    pltpu.async_copy(cemb_hbm.at[idx_p_v], rows_p_v.at[pl.ds(0, P_PER_W)],
                     psem).wait()

  wv = [wrow_v[0, pl.ds(j * L, L)] for j in range(D_VREGS)]

  @pl.when(wid < P_WORKERS)
  def _():
    _dot16(rows_p_v, 0, wv, out_p_v, 0)
    pltpu.sync_copy(out_p_v.at[pl.ds(0, P_PER_W)],
                    out_p_hbm.at[pl.ds(wid * P_PER_W, P_PER_W)])

  for j in range(N_CHUNKS):
    copies[j].wait()
    base = j * 128

    @plsc.parallel_loop(0, 128 // L, unroll=1)
    def _(g):
      row = base + g * L
      _dot16(rows_n_v, row, wv, out_n_v, row)

  pltpu.sync_copy(out_n_v, out_n_hbm.at[pl.ds(wid * N_PER_W, N_PER_W)])


@jax.jit
def _w2v(word, positive_contexts, negative_contexts, word_embeddings,
         context_embeddings):
  mesh = plsc.VectorSubcoreMesh(
      core_axis_name="c", subcore_axis_name="s", num_cores=NC,
      num_subcores=NS)
  neg2d = negative_contexts.reshape(N // 128, 128)
  out_p, out_n = pl.kernel(
      _w2v_body,
      out_type=(
          jax.ShapeDtypeStruct((P,), jnp.float32),
          jax.ShapeDtypeStruct((N,), jnp.float32),
      ),
      mesh=mesh,
      compiler_params=pltpu.CompilerParams(needs_layout_passes=False),
      scratch_types=[
          pltpu.VMEM((1,), jnp.int32),
          pltpu.VMEM((1, DIM), jnp.float32),
          pltpu.VMEM((N_CHUNKS, 128), jnp.int32),
          pltpu.VMEM((N_PER_W, DIM), jnp.float32),
          pltpu.VMEM((P_PER_W,), jnp.int32),
          pltpu.VMEM((L, DIM), jnp.float32),
          pltpu.VMEM((N_PER_W,), jnp.float32),
          pltpu.VMEM((L,), jnp.float32),
          pltpu.SemaphoreType.DMA,
          pltpu.SemaphoreType.DMA,
          pltpu.SemaphoreType.DMA,
      ],
  )(word, positive_contexts, neg2d, word_embeddings, context_embeddings)
  return out_p, out_n


def kernel(word, positive_contexts, negative_contexts, word_embeddings,
           context_embeddings):
  word = word.astype(jnp.int32)
  positive_contexts = positive_contexts.astype(jnp.int32)
  negative_contexts = negative_contexts.astype(jnp.int32)
  return _w2v(word, positive_contexts, negative_contexts, word_embeddings,
              context_embeddings)


# speedup vs baseline: 1.1030x; 1.1030x over previous
import functools

import jax
import jax.numpy as jnp
from jax import lax
from jax.experimental import pallas as pl
from jax.experimental.pallas import tpu as pltpu
from jax.experimental.pallas import tpu_sc as plsc

VOCAB = 100000
DIM = 128
P = 200
N = 16384

NC = 2
NS = 16
NW = NC * NS
L = 16

N_PER_W = N // NW
N_CHUNKS = N_PER_W // 128
P_PER_W = 8
P_WORKERS = P // P_PER_W

D_VREGS = DIM // L


def _sigmoid(v):
  return 1.0 / (1.0 + jnp.exp(-v))


def _dot16(rows_ref, row_base, wv, out_ref, out_base):
  lane = lax.iota(jnp.int32, L)
  out_v = jnp.zeros((L,), jnp.float32)
  for r in range(L):
    row = row_base + r
    p = [rows_ref[row, pl.ds(j * L, L)] * wv[j] for j in range(D_VREGS)]
    while len(p) > 1:
      p = [p[i] + p[i + 1] for i in range(0, len(p), 2)]
    s = jnp.sum(p[0])
    out_v = jnp.where(lane == r, s, out_v)
  out_ref[pl.ds(out_base, L)] = _sigmoid(out_v)


def _w2v_body(word_hbm, pos_hbm, neg_hbm, wemb_hbm, cemb_hbm,
              out_p_hbm, out_n_hbm,
              word_v, wrow_v, idx_n_v, rows_n_v, idx_p_v, rows_p_v,
              out_n_v, out_p_v, sem, psem, wsem):
  wid = lax.axis_index("s") * NC + lax.axis_index("c")

  pltpu.sync_copy(word_hbm, word_v)
  pltpu.async_copy(wemb_hbm.at[word_v], wrow_v, wsem).wait()

  pltpu.sync_copy(neg_hbm.at[pl.ds(wid * N_CHUNKS, N_CHUNKS)], idx_n_v)

  copies = []
  for j in range(N_CHUNKS):
    copies.append(pltpu.async_copy(
        cemb_hbm.at[idx_n_v.at[j]], rows_n_v.at[pl.ds(j * 128, 128)], sem))

  @pl.when(wid < P_WORKERS)
  def _():
    pltpu.sync_copy(pos_hbm.at[pl.ds(wid * P_PER_W, P_PER_W)], idx_p_v)
    pltpu.async_copy(cemb_hbm.at[idx_p_v], rows_p_v.at[pl.ds(0, P_PER_W)],
                     psem).wait()

  wv = [wrow_v[0, pl.ds(j * L, L)] for j in range(D_VREGS)]

  @pl.when(wid < P_WORKERS)
  def _():
    _dot16(rows_p_v, 0, wv, out_p_v, 0)
    pltpu.sync_copy(out_p_v.at[pl.ds(0, P_PER_W)],
                    out_p_hbm.at[pl.ds(wid * P_PER_W, P_PER_W)])

  for j in range(N_CHUNKS):
    copies[j].wait()

  def group_body(g, _):
    row = g * L
    _dot16(rows_n_v, row, wv, out_n_v, row)
    return 0

  lax.fori_loop(0, N_PER_W // L, group_body, 0)

  pltpu.sync_copy(out_n_v, out_n_hbm.at[pl.ds(wid * N_PER_W, N_PER_W)])


@jax.jit
def _w2v(word, positive_contexts, negative_contexts, word_embeddings,
         context_embeddings):
  mesh = plsc.VectorSubcoreMesh(
      core_axis_name="c", subcore_axis_name="s", num_cores=NC,
      num_subcores=NS)
  neg2d = negative_contexts.reshape(N // 128, 128)
  out_p, out_n = pl.kernel(
      _w2v_body,
      out_type=(
          jax.ShapeDtypeStruct((P,), jnp.float32),
          jax.ShapeDtypeStruct((N,), jnp.float32),
      ),
      mesh=mesh,
      compiler_params=pltpu.CompilerParams(needs_layout_passes=False),
      scratch_types=[
          pltpu.VMEM((1,), jnp.int32),
          pltpu.VMEM((1, DIM), jnp.float32),
          pltpu.VMEM((N_CHUNKS, 128), jnp.int32),
          pltpu.VMEM((N_PER_W, DIM), jnp.float32),
          pltpu.VMEM((P_PER_W,), jnp.int32),
          pltpu.VMEM((L, DIM), jnp.float32),
          pltpu.VMEM((N_PER_W,), jnp.float32),
          pltpu.VMEM((L,), jnp.float32),
          pltpu.SemaphoreType.DMA,
          pltpu.SemaphoreType.DMA,
          pltpu.SemaphoreType.DMA,
      ],
  )(word, positive_contexts, neg2d, word_embeddings, context_embeddings)
  return out_p, out_n


def kernel(word, positive_contexts, negative_contexts, word_embeddings,
           context_embeddings):
  word = word.astype(jnp.int32)
  positive_contexts = positive_contexts.astype(jnp.int32)
  negative_contexts = negative_contexts.astype(jnp.int32)
  return _w2v(word, positive_contexts, negative_contexts, word_embeddings,
              context_embeddings)


# speedup vs baseline: 1.1288x; 1.0234x over previous
import functools

import jax
import jax.numpy as jnp
from jax import lax
from jax.experimental import pallas as pl
from jax.experimental.pallas import tpu as pltpu
from jax.experimental.pallas import tpu_sc as plsc

VOCAB = 100000
DIM = 128
P = 200
N = 16384

NC = 2
NS = 16
NW = NC * NS
L = 16

N_PER_W = N // NW
N_CHUNKS = N_PER_W // 128
P_PER_W = 8
P_WORKERS = P // P_PER_W

D_VREGS = DIM // L


def _sigmoid(v):
  return 1.0 / (1.0 + jnp.exp(-v))


def _dot16(rows_ref, row_base, wv, out_ref, out_base):
  lane = lax.iota(jnp.int32, L)
  out_v = jnp.zeros((L,), jnp.float32)
  for r in range(L):
    row = row_base + r
    p = [rows_ref[row, pl.ds(j * L, L)] * wv[j] for j in range(D_VREGS)]
    while len(p) > 1:
      p = [p[i] + p[i + 1] for i in range(0, len(p), 2)]
    s = jnp.sum(p[0])
    out_v = jnp.where(lane == r, s, out_v)
  out_ref[pl.ds(out_base, L)] = _sigmoid(out_v)


def _w2v_body(word_hbm, pos_hbm, neg_hbm, wemb_hbm, cemb_hbm,
              out_p_hbm, out_n_hbm,
              word_v, wrow_v, idx_n_v, rows_n_v, idx_p_v,
              out_n_v, sem, psem, wsem):
  wid = lax.axis_index("s") * NC + lax.axis_index("c")
  has_pos = wid < P_WORKERS

  pltpu.sync_copy(neg_hbm.at[pl.ds(wid * N_CHUNKS, N_CHUNKS)], idx_n_v)
  copies = []
  for j in range(N_CHUNKS):
    copies.append(pltpu.async_copy(
        cemb_hbm.at[idx_n_v.at[j]], rows_n_v.at[pl.ds(j * 128, 128)], sem))

  @pl.when(has_pos)
  def _():
    pltpu.sync_copy(pos_hbm.at[pl.ds(wid * P_PER_W, P_PER_W)], idx_p_v)
    pltpu.async_copy(cemb_hbm.at[idx_p_v],
                     rows_n_v.at[pl.ds(N_PER_W, P_PER_W)], psem).wait()

  pltpu.sync_copy(word_hbm, word_v)
  pltpu.async_copy(wemb_hbm.at[word_v], wrow_v, wsem).wait()
  wv = [wrow_v[0, pl.ds(j * L, L)] for j in range(D_VREGS)]

  for j in range(N_CHUNKS):
    copies[j].wait()

  def group_body(g, _):
    row = g * L
    _dot16(rows_n_v, row, wv, out_n_v, row)
    return 0

  n_groups = N_PER_W // L + has_pos.astype(jnp.int32)
  lax.fori_loop(0, n_groups, group_body, 0)

  pltpu.sync_copy(out_n_v.at[pl.ds(0, N_PER_W)],
                  out_n_hbm.at[pl.ds(wid * N_PER_W, N_PER_W)])

  @pl.when(has_pos)
  def _():
    pltpu.sync_copy(out_n_v.at[pl.ds(N_PER_W, P_PER_W)],
                    out_p_hbm.at[pl.ds(wid * P_PER_W, P_PER_W)])


@jax.jit
def _w2v(word, positive_contexts, negative_contexts, word_embeddings,
         context_embeddings):
  mesh = plsc.VectorSubcoreMesh(
      core_axis_name="c", subcore_axis_name="s", num_cores=NC,
      num_subcores=NS)
  neg2d = negative_contexts.reshape(N // 128, 128)
  out_p, out_n = pl.kernel(
      _w2v_body,
      out_type=(
          jax.ShapeDtypeStruct((P,), jnp.float32),
          jax.ShapeDtypeStruct((N,), jnp.float32),
      ),
      mesh=mesh,
      compiler_params=pltpu.CompilerParams(needs_layout_passes=False),
      scratch_types=[
          pltpu.VMEM((1,), jnp.int32),
          pltpu.VMEM((1, DIM), jnp.float32),
          pltpu.VMEM((N_CHUNKS, 128), jnp.int32),
          pltpu.VMEM((N_PER_W + L, DIM), jnp.float32),
          pltpu.VMEM((P_PER_W,), jnp.int32),
          pltpu.VMEM((N_PER_W + L,), jnp.float32),
          pltpu.SemaphoreType.DMA,
          pltpu.SemaphoreType.DMA,
          pltpu.SemaphoreType.DMA,
      ],
  )(word, positive_contexts, neg2d, word_embeddings, context_embeddings)
  return out_p, out_n


def kernel(word, positive_contexts, negative_contexts, word_embeddings,
           context_embeddings):
  word = word.astype(jnp.int32)
  positive_contexts = positive_contexts.astype(jnp.int32)
  negative_contexts = negative_contexts.astype(jnp.int32)
  return _w2v(word, positive_contexts, negative_contexts, word_embeddings,
              context_embeddings)


# speedup vs baseline: 1.1369x; 1.0072x over previous
import functools

import jax
import jax.numpy as jnp
from jax import lax
from jax.experimental import pallas as pl
from jax.experimental.pallas import tpu as pltpu
from jax.experimental.pallas import tpu_sc as plsc

VOCAB = 100000
DIM = 128
P = 200
N = 16384

NC = 2
NS = 16
NW = NC * NS
L = 16

N_PER_W = N // NW
N_CHUNKS = N_PER_W // 128
P_PER_W = 8
P_WORKERS = P // P_PER_W

D_VREGS = DIM // L


def _sigmoid(v):
  return 1.0 / (1.0 + jnp.exp(-v))


def _dot16(rows_ref, row_base, wv, out_ref, out_base):
  lane = lax.iota(jnp.int32, L)
  out_v = jnp.zeros((L,), jnp.float32)
  for r in range(L):
    row = row_base + r
    p = [rows_ref[row, pl.ds(j * L, L)] * wv[j] for j in range(D_VREGS)]
    while len(p) > 1:
      p = [p[i] + p[i + 1] for i in range(0, len(p), 2)]
    s = jnp.sum(p[0])
    out_v = jnp.where(lane == r, s, out_v)
  out_ref[pl.ds(out_base, L)] = _sigmoid(out_v)


def _w2v_body(word_hbm, pos_hbm, neg_hbm, wemb_hbm, cemb_hbm,
              out_p_hbm, out_n_hbm,
              word_v, wrow_v, idx_n_v, rows_n_v, idx_p_v,
              out_n_v, sem, psem, wsem):
  wid = lax.axis_index("s") * NC + lax.axis_index("c")
  has_pos = wid < P_WORKERS

  pltpu.sync_copy(neg_hbm.at[pl.ds(wid * N_CHUNKS, N_CHUNKS)], idx_n_v)
  copies = []
  for j in range(N_CHUNKS):
    copies.append(pltpu.async_copy(
        cemb_hbm.at[idx_n_v.at[j]], rows_n_v.at[pl.ds(j * 128, 128)],
        sem[j]))

  @pl.when(has_pos)
  def _():
    pltpu.sync_copy(pos_hbm.at[pl.ds(wid * P_PER_W, P_PER_W)], idx_p_v)
    pltpu.async_copy(cemb_hbm.at[idx_p_v],
                     rows_n_v.at[pl.ds(N_PER_W, P_PER_W)], psem).wait()

  pltpu.sync_copy(word_hbm, word_v)
  pltpu.async_copy(wemb_hbm.at[word_v], wrow_v, wsem).wait()
  wv = [wrow_v[0, pl.ds(j * L, L)] for j in range(D_VREGS)]

  groups_per_chunk = 128 // L

  def group_body(g, _):
    for j in range(N_CHUNKS):
      @pl.when(g == j * groups_per_chunk)
      def _():
        copies[j].wait()
    row = g * L
    _dot16(rows_n_v, row, wv, out_n_v, row)
    return 0

  n_groups = N_PER_W // L + has_pos.astype(jnp.int32)
  lax.fori_loop(0, n_groups, group_body, 0)

  pltpu.sync_copy(out_n_v.at[pl.ds(0, N_PER_W)],
                  out_n_hbm.at[pl.ds(wid * N_PER_W, N_PER_W)])

  @pl.when(has_pos)
  def _():
    pltpu.sync_copy(out_n_v.at[pl.ds(N_PER_W, P_PER_W)],
                    out_p_hbm.at[pl.ds(wid * P_PER_W, P_PER_W)])


@jax.jit
def _w2v(word, positive_contexts, negative_contexts, word_embeddings,
         context_embeddings):
  mesh = plsc.VectorSubcoreMesh(
      core_axis_name="c", subcore_axis_name="s", num_cores=NC,
      num_subcores=NS)
  neg2d = negative_contexts.reshape(N // 128, 128)
  out_p, out_n = pl.kernel(
      _w2v_body,
      out_type=(
          jax.ShapeDtypeStruct((P,), jnp.float32),
          jax.ShapeDtypeStruct((N,), jnp.float32),
      ),
      mesh=mesh,
      compiler_params=pltpu.CompilerParams(needs_layout_passes=False),
      scratch_types=[
          pltpu.VMEM((1,), jnp.int32),
          pltpu.VMEM((1, DIM), jnp.float32),
          pltpu.VMEM((N_CHUNKS, 128), jnp.int32),
          pltpu.VMEM((N_PER_W + L, DIM), jnp.float32),
          pltpu.VMEM((P_PER_W,), jnp.int32),
          pltpu.VMEM((N_PER_W + L,), jnp.float32),
          [pltpu.SemaphoreType.DMA] * N_CHUNKS,
          pltpu.SemaphoreType.DMA,
          pltpu.SemaphoreType.DMA,
      ],
  )(word, positive_contexts, neg2d, word_embeddings, context_embeddings)
  return out_p, out_n


def kernel(word, positive_contexts, negative_contexts, word_embeddings,
           context_embeddings):
  word = word.astype(jnp.int32)
  positive_contexts = positive_contexts.astype(jnp.int32)
  negative_contexts = negative_contexts.astype(jnp.int32)
  return _w2v(word, positive_contexts, negative_contexts, word_embeddings,
              context_embeddings)
